# baseline (speedup 1.0000x reference)
import functools

import jax
import jax.numpy as jnp
from jax import lax
from jax.experimental import pallas as pl
from jax.experimental.pallas import tpu as pltpu
from jax.experimental.pallas import tpu_sc as plsc


def _dist_argmin_body(xb_ref, cb_ref, xsq_ref, cbsq_ref, idx_ref, loss_ref,
                      acc_ref):
    i = pl.program_id(0)
    xb = xb_ref[...]
    cb = cb_ref[...]
    x_sq = xsq_ref[...]
    cb_sq = cbsq_ref[...]
    k = cb.shape[0]
    ch = 256
    n_ch = k // ch
    xb2 = xb + xb
    best = None
    for c in range(n_ch):
        mm2_c = lax.dot_general(
            xb2, cb[c * ch:(c + 1) * ch, :], (((1,), (1,)), ((), ())),
            preferred_element_type=jnp.float32,
            precision=lax.Precision.DEFAULT)
        d_c = (x_sq + cb_sq[None, c * ch:(c + 1) * ch]) - mm2_c
        if best is None:
            best, bidx = d_c, jnp.zeros(d_c.shape, jnp.int32)
        else:
            upd = d_c < best
            best = jnp.where(upd, d_c, best)
            bidx = jnp.where(upd, jnp.int32(c), bidx)
    rowmin = jnp.min(best, axis=1, keepdims=True)
    lane = lax.broadcasted_iota(jnp.int32, best.shape, 1)
    cand = jnp.where(best == rowmin, bidx * ch + lane, jnp.int32(k))
    idx = jnp.min(cand, axis=1)
    idx_ref[...] = idx

    @pl.when(i == 0)
    def _():
        acc_ref[0] = 0.0

    acc_ref[0] += jnp.sum(rowmin)

    @pl.when(i == pl.num_programs(0) - 1)
    def _():
        loss_ref[0] = acc_ref[0]


def _dist_argmin(x_flat, codebook, x_sq, cb_sq):
    n, c = x_flat.shape
    k = codebook.shape[0]
    blk = 512
    grid = n // blk
    return pl.pallas_call(
        _dist_argmin_body,
        grid=(grid,),
        in_specs=[
            pl.BlockSpec((blk, c), lambda i: (i, 0)),
            pl.BlockSpec((k, c), lambda i: (0, 0)),
            pl.BlockSpec((blk, 1), lambda i: (i, 0)),
            pl.BlockSpec((k,), lambda i: (0,)),
        ],
        out_specs=[
            pl.BlockSpec((blk,), lambda i: (i,)),
            pl.BlockSpec(memory_space=pltpu.SMEM),
        ],
        out_shape=[
            jax.ShapeDtypeStruct((n,), jnp.int32),
            jax.ShapeDtypeStruct((1,), jnp.float32),
        ],
        scratch_shapes=[pltpu.SMEM((1,), jnp.float32)],
    )(x_flat, codebook, x_sq, cb_sq)


def _sc_gather(codebook, idx):
    info = plsc.get_sparse_core_info()
    nw = info.num_cores * info.num_subcores
    b = idx.shape[0]
    d = codebook.shape[1]
    b_per_w = b // nw
    ch = 128
    n_ch = b_per_w // ch
    mesh = plsc.VectorSubcoreMesh(core_axis_name="c", subcore_axis_name="s")

    @functools.partial(
        pl.kernel, mesh=mesh,
        out_type=jax.ShapeDtypeStruct((b, d), jnp.float32),
        scratch_types=[
            pltpu.VMEM((b_per_w,), jnp.int32),
            pltpu.VMEM((ch, d), jnp.float32),
            pltpu.VMEM((ch, d), jnp.float32),
            pltpu.SemaphoreType.DMA,
            pltpu.SemaphoreType.DMA,
        ],
    )
    def gk(cb_hbm, idx_hbm, out_hbm, idx_v, rows0, rows1, sem0, sem1):
        wid = lax.axis_index("s") * info.num_cores + lax.axis_index("c")
        base = wid * b_per_w
        pltpu.sync_copy(idx_hbm.at[pl.ds(base, b_per_w)], idx_v)
        bufs = (rows0, rows1)
        sems = (sem0, sem1)
        cps = [None, None]
        for j in range(n_ch):
            cps[j % 2] = pltpu.async_copy(
                cb_hbm.at[idx_v.at[pl.ds(j * ch, ch)]], bufs[j % 2], sems[j % 2])
            if j >= 1:
                cps[(j - 1) % 2].wait()
                pltpu.sync_copy(bufs[(j - 1) % 2],
                                out_hbm.at[pl.ds(base + (j - 1) * ch, ch)])
        cps[(n_ch - 1) % 2].wait()
        pltpu.sync_copy(bufs[(n_ch - 1) % 2],
                        out_hbm.at[pl.ds(base + (n_ch - 1) * ch, ch)])

    return gk(codebook, idx)


def kernel(x, codebook):
    b, c, h, w = x.shape
    n = b * h * w
    x_flat = jnp.transpose(x, (0, 2, 3, 1)).reshape(n, c)
    x_sq = jnp.sum(x_flat ** 2, axis=1, keepdims=True)
    cb_sq = jnp.sum(codebook ** 2, axis=1)
    half = n // 2
    idx1, ls1 = _dist_argmin(x_flat[:half], codebook, x_sq[:half], cb_sq)
    xq1 = _sc_gather(codebook, idx1)
    idx2, ls2 = _dist_argmin(x_flat[half:], codebook, x_sq[half:], cb_sq)
    xq2 = _sc_gather(codebook, idx2)
    idx = jnp.concatenate([idx1, idx2], axis=0)
    xq_flat = jnp.concatenate([xq1, xq2], axis=0)
    x_q = jnp.transpose(xq_flat.reshape(b, h, w, c), (0, 3, 1, 2))
    loss = (ls1[0] + ls2[0]) * (1.25 / (n * c))
    return (x_q, idx, loss)


# speedup vs baseline: 1.3446x; 1.3446x over previous
import functools

import jax
import jax.numpy as jnp
from jax import lax
from jax.experimental import pallas as pl
from jax.experimental.pallas import tpu as pltpu
from jax.experimental.pallas import tpu_sc as plsc


def _dist_argmin_body(xb_ref, cb_hbm_ref, xsq_ref, cbsq_ref, idx_ref, loss_ref,
                      acc_ref, cb_vmem, cb_sem):
    i = pl.program_id(0)

    @pl.when(i == 0)
    def _():
        pltpu.make_async_copy(cb_hbm_ref, cb_vmem, cb_sem).start()
        pltpu.make_async_copy(cb_hbm_ref, cb_vmem, cb_sem).wait()

    xb = xb_ref[...]
    cb = cb_vmem[...]
    x_sq = xsq_ref[...]
    cb_sq = cbsq_ref[...]
    k = cb.shape[0]
    ch = 256
    n_ch = k // ch
    xb2 = xb + xb
    best = None
    for c in range(n_ch):
        mm2_c = lax.dot_general(
            xb2, cb[c * ch:(c + 1) * ch, :], (((1,), (1,)), ((), ())),
            preferred_element_type=jnp.float32,
            precision=lax.Precision.DEFAULT)
        d_c = (x_sq + cb_sq[None, c * ch:(c + 1) * ch]) - mm2_c
        if best is None:
            best, bidx = d_c, jnp.zeros(d_c.shape, jnp.int32)
        else:
            upd = d_c < best
            best = jnp.where(upd, d_c, best)
            bidx = jnp.where(upd, jnp.int32(c), bidx)
    rowmin = jnp.min(best, axis=1, keepdims=True)
    lane = lax.broadcasted_iota(jnp.int32, best.shape, 1)
    cand = jnp.where(best == rowmin, bidx * ch + lane, jnp.int32(k))
    idx = jnp.min(cand, axis=1)
    idx_ref[...] = idx

    @pl.when(i == 0)
    def _():
        acc_ref[0] = 0.0

    acc_ref[0] += jnp.sum(rowmin)

    @pl.when(i == pl.num_programs(0) - 1)
    def _():
        loss_ref[0] = acc_ref[0]


def _dist_argmin(x_flat, codebook, x_sq, cb_sq):
    n, c = x_flat.shape
    k = codebook.shape[0]
    blk = 512
    grid = n // blk
    return pl.pallas_call(
        _dist_argmin_body,
        grid=(grid,),
        in_specs=[
            pl.BlockSpec((blk, c), lambda i: (i, 0)),
            pl.BlockSpec(memory_space=pltpu.HBM),
            pl.BlockSpec((blk, 1), lambda i: (i, 0)),
            pl.BlockSpec((k,), lambda i: (0,)),
        ],
        out_specs=[
            pl.BlockSpec((blk,), lambda i: (i,)),
            pl.BlockSpec(memory_space=pltpu.SMEM),
        ],
        out_shape=[
            jax.ShapeDtypeStruct((n,), jnp.int32),
            jax.ShapeDtypeStruct((1,), jnp.float32),
        ],
        scratch_shapes=[pltpu.SMEM((1,), jnp.float32),
                        pltpu.VMEM((k, c), jnp.float32),
                        pltpu.SemaphoreType.DMA],
    )(x_flat, codebook, x_sq, cb_sq)


def _sc_gather(codebook, idx):
    info = plsc.get_sparse_core_info()
    nw = info.num_cores * info.num_subcores
    b = idx.shape[0]
    d = codebook.shape[1]
    b_per_w = b // nw
    ch = 128
    n_ch = b_per_w // ch
    mesh = plsc.VectorSubcoreMesh(core_axis_name="c", subcore_axis_name="s")

    @functools.partial(
        pl.kernel, mesh=mesh,
        out_type=jax.ShapeDtypeStruct((b, d), jnp.float32),
        scratch_types=[
            pltpu.VMEM((b_per_w,), jnp.int32),
            pltpu.VMEM((ch, d), jnp.float32),
            pltpu.VMEM((ch, d), jnp.float32),
            pltpu.SemaphoreType.DMA,
            pltpu.SemaphoreType.DMA,
        ],
    )
    def gk(cb_hbm, idx_hbm, out_hbm, idx_v, rows0, rows1, sem0, sem1):
        wid = lax.axis_index("s") * info.num_cores + lax.axis_index("c")
        base = wid * b_per_w
        pltpu.sync_copy(idx_hbm.at[pl.ds(base, b_per_w)], idx_v)
        bufs = (rows0, rows1)
        sems = (sem0, sem1)
        cps = [None, None]
        for j in range(n_ch):
            cps[j % 2] = pltpu.async_copy(
                cb_hbm.at[idx_v.at[pl.ds(j * ch, ch)]], bufs[j % 2], sems[j % 2])
            if j >= 1:
                cps[(j - 1) % 2].wait()
                pltpu.sync_copy(bufs[(j - 1) % 2],
                                out_hbm.at[pl.ds(base + (j - 1) * ch, ch)])
        cps[(n_ch - 1) % 2].wait()
        pltpu.sync_copy(bufs[(n_ch - 1) % 2],
                        out_hbm.at[pl.ds(base + (n_ch - 1) * ch, ch)])

    return gk(codebook, idx)


def kernel(x, codebook):
    b, c, h, w = x.shape
    n = b * h * w
    x_flat = jnp.transpose(x, (0, 2, 3, 1)).reshape(n, c)
    x_sq = jnp.sum(x_flat ** 2, axis=1, keepdims=True)
    cb_sq = jnp.sum(codebook ** 2, axis=1)
    idx, loss_sum = _dist_argmin(x_flat, codebook, x_sq, cb_sq)
    xq_flat = _sc_gather(codebook, idx)
    x_q = jnp.transpose(xq_flat.reshape(b, h, w, c), (0, 3, 1, 2))
    loss = loss_sum[0] * (1.25 / (n * c))
    return (x_q, idx, loss)


# speedup vs baseline: 1.3567x; 1.0090x over previous
import functools

import jax
import jax.numpy as jnp
from jax import lax
from jax.experimental import pallas as pl
from jax.experimental.pallas import tpu as pltpu
from jax.experimental.pallas import tpu_sc as plsc


def _dist_argmin_body(xb_ref, cb_ref, xsq_ref, cbsq_ref, idx_ref, loss_ref,
                      acc_ref):
    i = pl.program_id(0)
    xb = xb_ref[...]
    cb = cb_ref[...]
    x_sq = xsq_ref[...]
    cb_sq = cbsq_ref[...]
    k = cb.shape[0]
    ch = 256
    n_ch = k // ch
    xb2 = xb + xb
    best = None
    for c in range(n_ch):
        mm2_c = lax.dot_general(
            xb2, cb[c * ch:(c + 1) * ch, :], (((1,), (1,)), ((), ())),
            preferred_element_type=jnp.float32,
            precision=lax.Precision.DEFAULT)
        d_c = (x_sq + cb_sq[None, c * ch:(c + 1) * ch]) - mm2_c
        if best is None:
            best, bidx = d_c, jnp.zeros(d_c.shape, jnp.int32)
        else:
            upd = d_c < best
            best = jnp.where(upd, d_c, best)
            bidx = jnp.where(upd, jnp.int32(c), bidx)
    rowmin = jnp.min(best, axis=1, keepdims=True)
    lane = lax.broadcasted_iota(jnp.int32, best.shape, 1)
    cand = jnp.where(best == rowmin, bidx * ch + lane, jnp.int32(k))
    idx = jnp.min(cand, axis=1)
    idx_ref[...] = idx

    @pl.when(i == 0)
    def _():
        acc_ref[0] = 0.0

    acc_ref[0] += jnp.sum(rowmin)

    @pl.when(i == pl.num_programs(0) - 1)
    def _():
        loss_ref[0] = acc_ref[0]


def _dist_argmin(x_flat, codebook, x_sq, cb_sq):
    n, c = x_flat.shape
    k = codebook.shape[0]
    blk = 512
    grid = n // blk
    return pl.pallas_call(
        _dist_argmin_body,
        grid=(grid,),
        in_specs=[
            pl.BlockSpec((blk, c), lambda i: (i, 0)),
            pl.BlockSpec((k, c), lambda i: (0, 0)),
            pl.BlockSpec((blk, 1), lambda i: (i, 0)),
            pl.BlockSpec((k,), lambda i: (0,)),
        ],
        out_specs=[
            pl.BlockSpec((blk,), lambda i: (i,)),
            pl.BlockSpec(memory_space=pltpu.SMEM),
        ],
        out_shape=[
            jax.ShapeDtypeStruct((n,), jnp.int32),
            jax.ShapeDtypeStruct((1,), jnp.float32),
        ],
        scratch_shapes=[pltpu.SMEM((1,), jnp.float32)],
    )(x_flat, codebook, x_sq, cb_sq)


def _sc_gather(codebook, idx):
    info = plsc.get_sparse_core_info()
    nw = info.num_cores * info.num_subcores
    b = idx.shape[0]
    d = codebook.shape[1]
    b_per_w = b // nw
    ch = 128
    n_ch = b_per_w // ch
    mesh = plsc.VectorSubcoreMesh(core_axis_name="c", subcore_axis_name="s")

    @functools.partial(
        pl.kernel, mesh=mesh,
        out_type=jax.ShapeDtypeStruct((b, d), jnp.float32),
        scratch_types=[
            pltpu.VMEM((b_per_w,), jnp.int32),
            pltpu.VMEM((ch, d), jnp.float32),
            pltpu.VMEM((ch, d), jnp.float32),
            pltpu.SemaphoreType.DMA,
            pltpu.SemaphoreType.DMA,
        ],
    )
    def gk(cb_hbm, idx_hbm, out_hbm, idx_v, rows0, rows1, sem0, sem1):
        wid = lax.axis_index("s") * info.num_cores + lax.axis_index("c")
        base = wid * b_per_w
        pltpu.sync_copy(idx_hbm.at[pl.ds(base, b_per_w)], idx_v)
        bufs = (rows0, rows1)
        sems = (sem0, sem1)
        cps = [None, None]
        for j in range(n_ch):
            cps[j % 2] = pltpu.async_copy(
                cb_hbm.at[idx_v.at[pl.ds(j * ch, ch)]], bufs[j % 2], sems[j % 2])
            if j >= 1:
                cps[(j - 1) % 2].wait()
                pltpu.sync_copy(bufs[(j - 1) % 2],
                                out_hbm.at[pl.ds(base + (j - 1) * ch, ch)])
        cps[(n_ch - 1) % 2].wait()
        pltpu.sync_copy(bufs[(n_ch - 1) % 2],
                        out_hbm.at[pl.ds(base + (n_ch - 1) * ch, ch)])

    return gk(codebook, idx)


def kernel(x, codebook):
    b, c, h, w = x.shape
    n = b * h * w
    x_flat = jnp.transpose(x, (0, 2, 3, 1)).reshape(n, c)
    x_sq = jnp.sum(x_flat ** 2, axis=1, keepdims=True)
    cb_sq = jnp.sum(codebook ** 2, axis=1)
    idx, loss_sum = _dist_argmin(x_flat, codebook, x_sq, cb_sq)
    xq_flat = _sc_gather(codebook, idx)
    x_q = jnp.transpose(xq_flat.reshape(b, h, w, c), (0, 3, 1, 2))
    loss = loss_sum[0] * (1.25 / (n * c))
    return (x_q, idx, loss)
